# baseline (speedup 1.0000x reference)
import functools

import jax
import jax.numpy as jnp
from jax import lax
from jax.experimental import pallas as pl
from jax.experimental.pallas import tpu as pltpu
from jax.experimental.pallas import tpu_sc as plsc

B = 16384
K = 20
H = 128
NW = 32
CHUNKS = 64
CB = 8
PAIRS = CB * K
PER_W = B // NW
IDX_ROWS = PER_W * K // 80
VSUB = PAIRS // 80
DEPTH = 4

_DNUMS = lax.GatherDimensionNumbers(
    offset_dims=(), collapsed_slice_dims=(0,), start_index_map=(0,))


def _perm(x, lane, s):
    idx = (lane ^ s).reshape(16, 1)
    return lax.gather(x, idx, _DNUMS, (1,),
                      mode=lax.GatherScatterMode.PROMISE_IN_BOUNDS)


def _combine(a, b, lane, s):
    m = (lane & s) == 0
    return jnp.where(m, a, _perm(b, lane, s)) + jnp.where(m, _perm(a, lane, s), b)


def _tree(vs, lane):
    s = 1
    while len(vs) > 1:
        vs = [_combine(vs[2 * i], vs[2 * i + 1], lane, s)
              for i in range(len(vs) // 2)]
        s *= 2
    return vs[0], s


CYCLES = CHUNKS // DEPTH


def _body(cid_hbm, ctx_hbm, u_hbm, v_hbm, out_hbm, cidx_v, kidx_v, *bufs):
    wid = lax.axis_index("s") * 2 + lax.axis_index("c")
    lane = lax.iota(jnp.int32, 16)
    slots = tuple(bufs[3 * b:3 * b + 3] for b in range(DEPTH))
    obufs = tuple(bufs[3 * DEPTH + 2 * o:3 * DEPTH + 2 * o + 2] for o in range(2))

    pltpu.sync_copy(cid_hbm.at[wid], cidx_v)
    pltpu.sync_copy(ctx_hbm.at[wid], kidx_v)

    def gather_descs(c, urows, vrows, gsem):
        cps = [pltpu.make_async_copy(u_hbm.at[cidx_v.at[c]], urows, gsem)]
        for j in range(VSUB):
            cps.append(pltpu.make_async_copy(
                v_hbm.at[kidx_v.at[c * VSUB + j]],
                vrows.at[pl.ds(80 * j, 80)], gsem))
        return cps

    def compute_chunk(urows_v, vrows_v, ostage_v, row0):
        def center_body(i, carry2):
            u = [urows_v[i, pl.ds(16 * t, 16)] for t in range(8)]
            accs = []
            for k in range(K):
                p = i * K + k
                acc = vrows_v[p, pl.ds(0, 16)] * u[0]
                for t in range(1, 8):
                    acc = acc + vrows_v[p, pl.ds(16 * t, 16)] * u[t]
                accs.append(acc)
            r16, _ = _tree(accs[:16], lane)
            e, s = _tree(accs[16:], lane)
            while s < 16:
                e = e + _perm(e, lane, s)
                s *= 2
            ostage_v[row0 + i, pl.ds(0, 16)] = r16
            ostage_v[row0 + i, pl.ds(16, 16)] = e
            return carry2

        lax.fori_loop(0, CB, center_body, 0, unroll=False)

    for b in range(DEPTH - 1):
        urows, vrows, gsem = slots[b]
        for cp in gather_descs(b, urows, vrows, gsem):
            cp.start()

    def ring_body(it2, carry):
        for o in range(2):
            it = 2 * it2 + o
            ostage, osem = obufs[o]

            @pl.when(it >= 2)
            def _():
                pltpu.make_async_copy(
                    ostage, out_hbm.at[wid * CYCLES + it], osem).wait()

            for b in range(DEPTH):
                c = DEPTH * it + b
                urows, vrows, gsem = slots[b]
                nurows, nvrows, ngsem = slots[(b + DEPTH - 1) % DEPTH]
                for cp in gather_descs(c, urows, vrows, gsem):
                    cp.wait()

                @pl.when(c + DEPTH - 1 < CHUNKS)
                def _():
                    for cp in gather_descs(c + DEPTH - 1, nurows, nvrows,
                                           ngsem):
                        cp.start()

                compute_chunk(urows, vrows, ostage, b * CB)

            pltpu.make_async_copy(
                ostage, out_hbm.at[wid * CYCLES + it], osem).start()
        return carry

    lax.fori_loop(0, CYCLES // 2, ring_body, 0, unroll=False)

    for o in range(2):
        ostage, osem = obufs[o]
        pltpu.make_async_copy(
            ostage, out_hbm.at[wid * CYCLES + (CYCLES - 2 + o)], osem).wait()


_sc_kernel = functools.partial(
    pl.kernel,
    out_type=jax.ShapeDtypeStruct((NW * (CHUNKS // DEPTH), DEPTH * CB, 32),
                                  jnp.float32),
    mesh=plsc.VectorSubcoreMesh(core_axis_name="c", subcore_axis_name="s"),
    scratch_types=(
        [
            pltpu.VMEM((CHUNKS, CB), jnp.int32),
            pltpu.VMEM((IDX_ROWS, 80), jnp.int32),
        ]
        + [
            t
            for _ in range(DEPTH)
            for t in (
                pltpu.VMEM((CB, H), jnp.float32),
                pltpu.VMEM((PAIRS, H), jnp.float32),
                pltpu.SemaphoreType.DMA,
            )
        ]
        + [
            t
            for _ in range(2)
            for t in (
                pltpu.VMEM((DEPTH * CB, 32), jnp.float32),
                pltpu.SemaphoreType.DMA,
            )
        ]
    ),
)(_body)


def kernel(center_ids, context_neg_ids, U, V):
    cid = center_ids.reshape(-1).astype(jnp.int32).reshape(NW, CHUNKS, CB)
    ctx = context_neg_ids.reshape(-1).astype(jnp.int32).reshape(NW, IDX_ROWS, 80)
    out = _sc_kernel(cid, ctx, U, V)
    return out.reshape(B, 32)[:, :K]


# speedup vs baseline: 1.0283x; 1.0283x over previous
import functools

import jax
import jax.numpy as jnp
from jax import lax
from jax.experimental import pallas as pl
from jax.experimental.pallas import tpu as pltpu
from jax.experimental.pallas import tpu_sc as plsc

B = 16384
K = 20
H = 128
NW = 32
CHUNKS = 64
CB = 8
PAIRS = CB * K
PER_W = B // NW
IDX_ROWS = PER_W * K // 80
VSUB = PAIRS // 80
DEPTH = 4

_DNUMS = lax.GatherDimensionNumbers(
    offset_dims=(), collapsed_slice_dims=(0,), start_index_map=(0,))


def _perm(x, lane, s):
    idx = (lane ^ s).reshape(16, 1)
    return lax.gather(x, idx, _DNUMS, (1,),
                      mode=lax.GatherScatterMode.PROMISE_IN_BOUNDS)


def _combine(a, b, lane, s):
    m = (lane & s) == 0
    return jnp.where(m, a, _perm(b, lane, s)) + jnp.where(m, _perm(a, lane, s), b)


def _tree(vs, lane):
    s = 1
    while len(vs) > 1:
        vs = [_combine(vs[2 * i], vs[2 * i + 1], lane, s)
              for i in range(len(vs) // 2)]
        s *= 2
    return vs[0], s


def _body(cid_hbm, ctx_hbm, u_hbm, v_hbm, out_hbm, cidx_v, kidx_v, *bufs):
    wid = lax.axis_index("s") * 2 + lax.axis_index("c")
    lane = lax.iota(jnp.int32, 16)
    slots = tuple(bufs[5 * b:5 * b + 5] for b in range(DEPTH))

    pltpu.sync_copy(cid_hbm.at[wid], cidx_v)
    pltpu.sync_copy(ctx_hbm.at[wid], kidx_v)

    def gather_descs(c, urows, vrows, gsem):
        cps = [pltpu.make_async_copy(u_hbm.at[cidx_v.at[c]], urows, gsem)]
        for j in range(VSUB):
            cps.append(pltpu.make_async_copy(
                v_hbm.at[kidx_v.at[c * VSUB + j]],
                vrows.at[pl.ds(80 * j, 80)], gsem))
        return cps

    def compute_chunk(urows_v, vrows_v, ostage_v):
        def center_body(i, carry2):
            u = [urows_v[i, pl.ds(16 * t, 16)] for t in range(8)]
            accs = []
            for k in range(K):
                p = i * K + k
                acc = vrows_v[p, pl.ds(0, 16)] * u[0]
                for t in range(1, 8):
                    acc = acc + vrows_v[p, pl.ds(16 * t, 16)] * u[t]
                accs.append(acc)
            r16, _ = _tree(accs[:16], lane)
            e, s = _tree(accs[16:], lane)
            while s < 16:
                e = e + _perm(e, lane, s)
                s *= 2
            ostage_v[i, pl.ds(0, 16)] = r16
            ostage_v[i, pl.ds(16, 16)] = e
            return carry2

        lax.fori_loop(0, CB, center_body, 0, unroll=False)

    for b in range(DEPTH - 1):
        urows, vrows, _, gsem, _ = slots[b]
        for cp in gather_descs(b, urows, vrows, gsem):
            cp.start()

    def ring_body(it, carry):
        for b in range(DEPTH):
            c = DEPTH * it + b
            urows, vrows, ostage, gsem, osem = slots[b]
            nurows, nvrows, _, ngsem, _ = slots[(b + DEPTH - 1) % DEPTH]
            for cp in gather_descs(c, urows, vrows, gsem):
                cp.wait()

            @pl.when(c + DEPTH - 1 < CHUNKS)
            def _():
                for cp in gather_descs(c + DEPTH - 1, nurows, nvrows, ngsem):
                    cp.start()

            @pl.when(it > 0)
            def _():
                pltpu.make_async_copy(
                    ostage, out_hbm.at[wid * CHUNKS + c], osem).wait()

            compute_chunk(urows, vrows, ostage)
            pltpu.make_async_copy(
                ostage, out_hbm.at[wid * CHUNKS + c], osem).start()
        return carry

    lax.fori_loop(0, CHUNKS // DEPTH, ring_body, 0, unroll=False)

    for b in range(DEPTH):
        _, _, ostage, _, osem = slots[b]
        pltpu.make_async_copy(
            ostage, out_hbm.at[wid * CHUNKS + (CHUNKS - DEPTH + b)], osem).wait()


_sc_kernel = functools.partial(
    pl.kernel,
    out_type=jax.ShapeDtypeStruct((NW * CHUNKS, CB, 32), jnp.float32),
    mesh=plsc.VectorSubcoreMesh(core_axis_name="c", subcore_axis_name="s"),
    scratch_types=(
        [
            pltpu.VMEM((CHUNKS, CB), jnp.int32),
            pltpu.VMEM((IDX_ROWS, 80), jnp.int32),
        ]
        + [
            t
            for _ in range(DEPTH)
            for t in (
                pltpu.VMEM((CB, H), jnp.float32),
                pltpu.VMEM((PAIRS, H), jnp.float32),
                pltpu.VMEM((CB, 32), jnp.float32),
                pltpu.SemaphoreType.DMA,
                pltpu.SemaphoreType.DMA,
            )
        ]
    ),
)(_body)


def kernel(center_ids, context_neg_ids, U, V):
    cid = center_ids.reshape(-1).astype(jnp.int32).reshape(NW, CHUNKS, CB)
    ctx = context_neg_ids.reshape(-1).astype(jnp.int32).reshape(NW, IDX_ROWS, 80)
    out = _sc_kernel(cid, ctx, U, V)
    return out.reshape(B, 32)[:, :K]
